# baseline (speedup 1.0000x reference)
import jax
import jax.numpy as jnp
from jax import lax
from jax.experimental import pallas as pl
from jax.experimental.pallas import tpu as pltpu
from jax.experimental.pallas import tpu_sc as plsc

_VOCAB = 1000000
_D = 64
_B = 4096
_N = 200
_TOT = _B * _N
_NW = 32
_PER_W = _TOT // _NW
_SCHUNK = 128
_NSTREAM = _PER_W // _SCHUNK


def _sc_body(xflat_hbm, tab_hbm, posd_hbm, out_hbm, idx1_v, posd_v, rows_v,
             sem):
    wid = lax.axis_index("s") * 2 + lax.axis_index("c")
    base = wid * _PER_W
    pltpu.sync_copy(xflat_hbm.at[pl.ds(base, _PER_W)], idx1_v)
    pltpu.sync_copy(posd_hbm, posd_v)

    def stream_body(s, carry):
        pltpu.async_copy(
            tab_hbm.at[idx1_v.at[pl.ds(s * _SCHUNK, _SCHUNK)]], rows_v, sem
        ).wait()
        off64 = lax.rem(s * _SCHUNK, _N) * _D

        def row_body(j, c2):
            iv = plsc.load_gather(
                idx1_v, [lax.broadcast(s * _SCHUNK + j, (16,))]
            )
            m = jnp.where(iv != 0, 1.0, 0.0).astype(jnp.float32)
            pbase = off64 + j * _D
            for g in range(4):
                val = rows_v[j, pl.ds(g * 16, 16)]
                p = posd_v[pl.ds(pbase + g * 16, 16)]
                rows_v[j, pl.ds(g * 16, 16)] = val * m + p
            return c2

        lax.fori_loop(0, _SCHUNK, row_body, 0, unroll=4)
        pltpu.sync_copy(rows_v, out_hbm.at[pl.ds(base + s * _SCHUNK, _SCHUNK)])
        return carry

    lax.fori_loop(0, _NSTREAM, stream_body, 0)


def kernel(X, table, pos):
    xflat = X.reshape(_TOT)
    pm = jnp.broadcast_to(pos.astype(jnp.float32).reshape(_N, 1), (_N, _D))
    posd = jnp.concatenate([pm, pm], axis=0).reshape(2 * _N * _D)
    k = pl.kernel(
        _sc_body,
        mesh=plsc.VectorSubcoreMesh(core_axis_name="c", subcore_axis_name="s"),
        compiler_params=pltpu.CompilerParams(
            needs_layout_passes=False, use_tc_tiling_on_sc=False
        ),
        out_type=jax.ShapeDtypeStruct((_TOT, _D), jnp.float32),
        scratch_types=[
            pltpu.VMEM((_PER_W,), jnp.int32),
            pltpu.VMEM((2 * _N * _D,), jnp.float32),
            pltpu.VMEM((_SCHUNK, _D), jnp.float32),
            pltpu.SemaphoreType.DMA,
        ],
    )
    out = k(xflat, table, posd)
    return out.reshape(_B, _N, _D)


# speedup vs baseline: 1.2571x; 1.2571x over previous
import jax
import jax.numpy as jnp
from jax import lax
from jax.experimental import pallas as pl
from jax.experimental.pallas import tpu as pltpu
from jax.experimental.pallas import tpu_sc as plsc

_VOCAB = 1000000
_D = 64
_B = 4096
_N = 200
_TOT = _B * _N
_NW = 32
_PER_W = _TOT // _NW
_SCHUNK = 128
_NSTREAM = _PER_W // _SCHUNK
_NBUF = 4
_NGRP = _NSTREAM // _NBUF


def _bc16(x):
    return lax.broadcast(x, (16,))


def _sc_body(xflat_hbm, tab_hbm, posd_hbm, out_hbm, idx1_v, posd_v, gbuf,
             wbuf, gsem, wsem):
    wid = lax.axis_index("s") * 2 + lax.axis_index("c")
    base = wid * _PER_W
    pltpu.sync_copy(xflat_hbm.at[pl.ds(base, _PER_W)], idx1_v)
    pltpu.sync_copy(posd_hbm, posd_v)

    def start_gather(s, b):
        pltpu.async_copy(
            tab_hbm.at[idx1_v.at[pl.ds(s * _SCHUNK, _SCHUNK)]],
            gbuf.at[b],
            gsem.at[b],
        )

    for b in range(_NBUF):
        start_gather(b, b)

    def grp_body(g, carry):
        for b in range(_NBUF):
            s = g * _NBUF + b
            pltpu.make_async_copy(
                tab_hbm.at[idx1_v.at[pl.ds(s * _SCHUNK, _SCHUNK)]],
                gbuf.at[b],
                gsem.at[b],
            ).wait()

            @pl.when(g > 0)
            def _():
                pltpu.make_async_copy(
                    wbuf.at[b],
                    out_hbm.at[pl.ds(base + s * _SCHUNK, _SCHUNK)],
                    wsem.at[b],
                ).wait()

            off = lax.rem(s * _SCHUNK, _N)

            def q_body(q, c2):
                r0 = q * 16
                for r in range(16):
                    iv = plsc.load_gather(
                        idx1_v, [_bc16(s * _SCHUNK + r0 + r)]
                    )
                    pv = plsc.load_gather(posd_v, [_bc16(off + r0 + r)])
                    m = jnp.where(iv != 0, 1.0, 0.0).astype(jnp.float32)
                    for gg in range(4):
                        val = gbuf[b, r0 + r, pl.ds(gg * 16, 16)]
                        wbuf[b, r0 + r, pl.ds(gg * 16, 16)] = val * m + pv
                return c2

            lax.fori_loop(0, _SCHUNK // 16, q_body, 0)

            pltpu.async_copy(
                wbuf.at[b],
                out_hbm.at[pl.ds(base + s * _SCHUNK, _SCHUNK)],
                wsem.at[b],
            )

            @pl.when(g < _NGRP - 1)
            def _():
                start_gather(s + _NBUF, b)
        return carry

    lax.fori_loop(0, _NGRP, grp_body, 0)

    for b in range(_NBUF):
        s = (_NGRP - 1) * _NBUF + b
        pltpu.make_async_copy(
            wbuf.at[b],
            out_hbm.at[pl.ds(base + s * _SCHUNK, _SCHUNK)],
            wsem.at[b],
        ).wait()


def kernel(X, table, pos):
    xflat = X.reshape(_TOT)
    p1 = pos.astype(jnp.float32).reshape(_N)
    posd = jnp.concatenate([p1, p1])
    k = pl.kernel(
        _sc_body,
        mesh=plsc.VectorSubcoreMesh(core_axis_name="c", subcore_axis_name="s"),
        compiler_params=pltpu.CompilerParams(
            needs_layout_passes=False, use_tc_tiling_on_sc=False
        ),
        out_type=jax.ShapeDtypeStruct((_TOT, _D), jnp.float32),
        scratch_types=[
            pltpu.VMEM((_PER_W,), jnp.int32),
            pltpu.VMEM((2 * _N,), jnp.float32),
            pltpu.VMEM((_NBUF, _SCHUNK, _D), jnp.float32),
            pltpu.VMEM((_NBUF, _SCHUNK, _D), jnp.float32),
            pltpu.SemaphoreType.DMA((_NBUF,)),
            pltpu.SemaphoreType.DMA((_NBUF,)),
        ],
    )
    out = k(xflat, table, posd)
    return out.reshape(_B, _N, _D)
